# Initial kernel scaffold; baseline (speedup 1.0000x reference)
import jax
import jax.numpy as jnp
from jax.experimental import pallas as pl


def kernel(x, edge_index, W_enc, b_enc, W_dec, b_dec):
    raise NotImplementedError("write your pallas kernel here")


# speedup vs baseline: 4.7717x; 4.7717x over previous
import functools

import jax
import jax.numpy as jnp
from jax import lax
from jax.experimental import pallas as pl
from jax.experimental.pallas import tpu as pltpu
from jax.experimental.pallas import tpu_sc as plsc

N_NODES = 10000
N_EDGES = 320000
D_FEAT = 128
D_HID = 64

NC = 2
NS = 16
NW = NC * NS
EPT = N_EDGES // NS
CHUNK = 80
NCH = EPT // CHUNK
NPAD = 10240
RPT = NPAD // NS
DH2 = D_FEAT // 2


def _sc_aggregate(x64, src3, dst3):
    mesh = plsc.VectorSubcoreMesh(core_axis_name="c", subcore_axis_name="s")

    @functools.partial(
        pl.kernel,
        out_type=[
            jax.ShapeDtypeStruct((NC, NPAD, DH2), jnp.float32),
            jax.ShapeDtypeStruct((NS, NPAD), jnp.float32),
        ],
        mesh=mesh,
        compiler_params=pltpu.CompilerParams(needs_layout_passes=False, use_tc_tiling_on_sc=False),
        scratch_types=[
            pltpu.VMEM((NCH, CHUNK), jnp.int32),
            pltpu.VMEM((NCH, CHUNK), jnp.int32),
            pltpu.VMEM((CHUNK, DH2), jnp.float32),
            pltpu.VMEM((NPAD,), jnp.float32),
            pltpu.VMEM_SHARED((NPAD, DH2), jnp.float32),
            pltpu.SemaphoreType.DMA,
        ],
    )
    def k(x_hbm, src_hbm, dst_hbm, agg_out, deg_out,
          src_v, dst_v, rows_v, degt_v, agg_sh, gsem):
        cid = lax.axis_index("c")
        sid = lax.axis_index("s")
        z16 = jnp.zeros((16,), jnp.float32)
        ones16 = jnp.ones((16,), jnp.float32)

        pltpu.sync_copy(src_hbm.at[sid], src_v)
        pltpu.sync_copy(dst_hbm.at[sid], dst_v)

        cvec = jnp.full((16,), cid, jnp.int32)
        def xform(j, c):
            for q in range(CHUNK // 16):
                sl = pl.ds(q * 16, 16)
                src_v[j, sl] = src_v[j, sl] * 2 + cvec
            return c
        lax.fori_loop(0, NCH, xform, 0)

        def zdeg(i, c):
            degt_v[pl.ds(i * 16, 16)] = z16
            return c
        def zrow(r, c):
            for q in range(DH2 // 16):
                rows_v[r, pl.ds(q * 16, 16)] = z16
            return c
        lax.fori_loop(0, CHUNK, zrow, 0)
        lax.fori_loop(0, NPAD // 16, zdeg, 0)
        def zsh(q, c):
            pltpu.sync_copy(rows_v.at[pl.ds(0, 80)],
                            agg_sh.at[pl.ds(sid * RPT + q * 80, 80)])
            return c
        lax.fori_loop(0, RPT // 80, zsh, 0)
        plsc.subcore_barrier()

        nfull = CHUNK // 16

        def body(j, carry):
            pltpu.async_copy(x_hbm.at[src_v.at[j]], rows_v, gsem).wait()
            pltpu.sync_copy(rows_v, agg_sh.at[dst_v.at[j]], add=True)

            @pl.when(cid == 0)
            def _():
                for q in range(nfull):
                    idx16 = dst_v[j, pl.ds(q * 16, 16)]
                    plsc.addupdate_scatter(degt_v, [idx16], ones16)
            return carry

        lax.fori_loop(0, NCH, body, 0)

        @pl.when(cid == 0)
        def _():
            pltpu.sync_copy(degt_v, deg_out.at[sid])
        plsc.subcore_barrier()

        sl = pl.ds(sid * RPT, RPT)
        pltpu.sync_copy(agg_sh.at[sl], agg_out.at[cid, sl])

    return k(x64, src3, dst3)


BLK = 1000


def _tc_dense(agg2, deg16, W_enc, b_enc, W_dec, b_dec):
    nblk = N_NODES // BLK

    def body(agg_ref, deg_ref, we_ref, be_ref, wd_ref, bd_ref,
             recon_ref, bott_ref, loss_ref, acc_ref):
        i = pl.program_id(0)
        a = jnp.concatenate([agg_ref[0], agg_ref[1]], axis=1)
        d = jnp.sum(deg_ref[...], axis=0)
        h = a / jnp.maximum(d, 1.0)
        e = jnp.maximum(
            jnp.dot(h, we_ref[...], preferred_element_type=jnp.float32)
            + be_ref[...], 0.0)
        r = jnp.maximum(
            jnp.dot(e, wd_ref[...], preferred_element_type=jnp.float32)
            + bd_ref[...], 0.0)
        recon_ref[...] = r
        bott_ref[...] = e
        df = r - h
        ps = jnp.sum(df * df)

        @pl.when(i == 0)
        def _():
            acc_ref[0, 0] = 0.0

        acc_ref[0, 0] = acc_ref[0, 0] + ps

        @pl.when(i == nblk - 1)
        def _():
            loss_ref[...] = jnp.full(
                (1, 1), acc_ref[0, 0] * (1.0 / (N_NODES * D_FEAT)), jnp.float32)

    recon, bott, loss = pl.pallas_call(
        body,
        grid=(nblk,),
        in_specs=[
            pl.BlockSpec((NC, BLK, DH2), lambda i: (0, i, 0)),
            pl.BlockSpec((NS, BLK, 1), lambda i: (0, i, 0)),
            pl.BlockSpec((D_FEAT, D_HID), lambda i: (0, 0)),
            pl.BlockSpec((1, D_HID), lambda i: (0, 0)),
            pl.BlockSpec((D_HID, D_FEAT), lambda i: (0, 0)),
            pl.BlockSpec((1, D_FEAT), lambda i: (0, 0)),
        ],
        out_specs=[
            pl.BlockSpec((BLK, D_FEAT), lambda i: (i, 0)),
            pl.BlockSpec((BLK, D_HID), lambda i: (i, 0)),
            pl.BlockSpec((1, 1), lambda i: (0, 0)),
        ],
        out_shape=[
            jax.ShapeDtypeStruct((N_NODES, D_FEAT), jnp.float32),
            jax.ShapeDtypeStruct((N_NODES, D_HID), jnp.float32),
            jax.ShapeDtypeStruct((1, 1), jnp.float32),
        ],
        scratch_shapes=[pltpu.SMEM((1, 1), jnp.float32)],
    )(agg2, deg16, W_enc, b_enc.reshape(1, D_HID), W_dec, b_dec.reshape(1, D_FEAT))
    return recon, bott, loss[0, 0]


def kernel(x, edge_index, W_enc, b_enc, W_dec, b_dec):
    ei = edge_index.astype(jnp.int32)
    src3 = ei[0].reshape(NS, NCH, CHUNK)
    dst3 = ei[1].reshape(NS, NCH, CHUNK)
    x64 = x.reshape(2 * N_NODES, DH2)
    agg2, deg16 = _sc_aggregate(x64, src3, dst3)
    deg16 = deg16.reshape(NS, NPAD, 1)
    recon, bott, loss = _tc_dense(agg2, deg16, W_enc, b_enc, W_dec, b_dec)
    return (recon, bott, loss)
